# Initial kernel scaffold; baseline (speedup 1.0000x reference)
import functools

import jax
import jax.numpy as jnp
from jax import lax
from jax.experimental import pallas as pl
from jax.experimental.pallas import tpu as pltpu
from jax.experimental.pallas import tpu_sc as plsc

VOCAB = 100000
DIM = 128
B = 16384
NC = 2
NS = 16
NW = NC * NS
BPW = B // NW
CHUNK = 128
NCHUNK = BPW // CHUNK
LANE = 16
GROUPS = CHUNK // LANE


def _mesh():
    return plsc.VectorSubcoreMesh(
        core_axis_name="c", subcore_axis_name="s", num_cores=NC, num_subcores=NS
    )


@functools.partial(
    pl.kernel,
    out_type=jax.ShapeDtypeStruct((B,), jnp.float32),
    mesh=_mesh(),
    scratch_types=[
        pltpu.VMEM((BPW,), jnp.int32),
        pltpu.VMEM((BPW,), jnp.int32),
        pltpu.VMEM((BPW,), jnp.float32),
        pltpu.VMEM((BPW,), jnp.float32),
        pltpu.VMEM((CHUNK, DIM), jnp.float32),
        pltpu.VMEM((CHUNK, DIM), jnp.float32),
        pltpu.VMEM((BPW,), jnp.float32),
        pltpu.SemaphoreType.DMA,
        pltpu.SemaphoreType.DMA,
        pltpu.SemaphoreType.DMA,
        pltpu.SemaphoreType.DMA,
    ],
)
def _glove_sc(i_hbm, j_hbm, emb_hbm, ctx_hbm, bias_hbm, cbias_hbm, out_hbm,
              idx_i, idx_j, bi, bj, wi, wj, outv,
              sem_wi, sem_wj, sem_bi, sem_bj):
    wid = lax.axis_index("s") * NC + lax.axis_index("c")
    base = wid * BPW

    pltpu.sync_copy(i_hbm.at[pl.ds(base, BPW)], idx_i)
    pltpu.sync_copy(j_hbm.at[pl.ds(base, BPW)], idx_j)

    lane = lax.iota(jnp.int32, (LANE,))

    for c in range(NCHUNK):
        off = c * CHUNK
        ii = idx_i.at[pl.ds(off, CHUNK)]
        jj = idx_j.at[pl.ds(off, CHUNK)]
        cw_i = pltpu.async_copy(emb_hbm.at[ii], wi, sem_wi)
        cw_j = pltpu.async_copy(ctx_hbm.at[jj], wj, sem_wj)
        cb_i = pltpu.async_copy(bias_hbm.at[ii], bi.at[pl.ds(off, CHUNK)], sem_bi)
        cb_j = pltpu.async_copy(cbias_hbm.at[jj], bj.at[pl.ds(off, CHUNK)], sem_bj)
        cw_i.wait()
        cw_j.wait()
        cb_i.wait()
        cb_j.wait()

        def group(g, _, off=off):
            rbase = g * LANE
            acc = jnp.zeros((LANE,), jnp.float32)
            for p in range(LANE):
                r = rbase + p
                s = wi[r, pl.ds(0, LANE)] * wj[r, pl.ds(0, LANE)]
                for k in range(1, DIM // LANE):
                    s = s + wi[r, pl.ds(k * LANE, LANE)] * wj[r, pl.ds(k * LANE, LANE)]
                tot = jnp.sum(s)
                acc = jnp.where(lane == p, tot, acc)
            ob = off + rbase
            outv[pl.ds(ob, LANE)] = (
                acc + bi[pl.ds(ob, LANE)] + bj[pl.ds(ob, LANE)]
            )
            return 0

        lax.fori_loop(0, GROUPS, group, 0)

    pltpu.sync_copy(outv, out_hbm.at[pl.ds(base, BPW)])


def kernel(i, j, embedding, context_embedding, bias, context_bias):
    return _glove_sc(
        jnp.asarray(i, jnp.int32),
        jnp.asarray(j, jnp.int32),
        embedding,
        context_embedding,
        bias,
        context_bias,
    )


# speedup vs baseline: 1.1898x; 1.1898x over previous
import functools

import jax
import jax.numpy as jnp
from jax import lax
from jax.experimental import pallas as pl
from jax.experimental.pallas import tpu as pltpu
from jax.experimental.pallas import tpu_sc as plsc

VOCAB = 100000
DIM = 128
B = 16384
NC = 2
NS = 16
NW = NC * NS
BPW = B // NW
CHUNK = 128
NCHUNK = BPW // CHUNK
LANE = 16
GROUPS = CHUNK // LANE


def _mesh():
    return plsc.VectorSubcoreMesh(
        core_axis_name="c", subcore_axis_name="s", num_cores=NC, num_subcores=NS
    )


@functools.partial(
    pl.kernel,
    out_type=jax.ShapeDtypeStruct((B,), jnp.float32),
    mesh=_mesh(),
    scratch_types=[
        pltpu.VMEM((BPW,), jnp.int32),
        pltpu.VMEM((BPW,), jnp.int32),
        pltpu.VMEM((BPW,), jnp.float32),
        pltpu.VMEM((BPW,), jnp.float32),
        pltpu.VMEM((CHUNK, DIM), jnp.float32),
        pltpu.VMEM((CHUNK, DIM), jnp.float32),
        pltpu.VMEM((BPW,), jnp.float32),
        pltpu.SemaphoreType.DMA,
        pltpu.SemaphoreType.DMA,
        pltpu.SemaphoreType.DMA,
        pltpu.SemaphoreType.DMA,
    ],
)
def _glove_sc(i_hbm, j_hbm, emb_hbm, ctx_hbm, bias_hbm, cbias_hbm, out_hbm,
              idx_i, idx_j, bi, bj, wi, wj, outv,
              sem_wi, sem_wj, sem_bi, sem_bj):
    wid = lax.axis_index("s") * NC + lax.axis_index("c")
    base = wid * BPW

    pltpu.sync_copy(i_hbm.at[pl.ds(base, BPW)], idx_i)
    pltpu.sync_copy(j_hbm.at[pl.ds(base, BPW)], idx_j)

    lane = lax.iota(jnp.int32, LANE)
    perms = [lane ^ k for k in (8, 4, 2, 1)]

    for c in range(NCHUNK):
        off = c * CHUNK
        ii = idx_i.at[pl.ds(off, CHUNK)]
        jj = idx_j.at[pl.ds(off, CHUNK)]
        cw_i = pltpu.async_copy(emb_hbm.at[ii], wi, sem_wi)
        cw_j = pltpu.async_copy(ctx_hbm.at[jj], wj, sem_wj)
        cb_i = pltpu.async_copy(bias_hbm.at[ii], bi.at[pl.ds(off, CHUNK)], sem_bi)
        cb_j = pltpu.async_copy(cbias_hbm.at[jj], bj.at[pl.ds(off, CHUNK)], sem_bj)
        cw_i.wait()
        cw_j.wait()
        cb_i.wait()
        cb_j.wait()

        def group(g, _, off=off):
            rbase = g * LANE
            acc = jnp.zeros((LANE,), jnp.float32)
            for p in range(LANE):
                r = rbase + p
                s = wi[r, pl.ds(0, LANE)] * wj[r, pl.ds(0, LANE)]
                for k in range(1, DIM // LANE):
                    s = s + wi[r, pl.ds(k * LANE, LANE)] * wj[r, pl.ds(k * LANE, LANE)]
                for perm in perms:
                    s = s + s.at[perm].get(mode="promise_in_bounds")
                acc = jnp.where(lane == p, s, acc)
            ob = off + rbase
            outv[pl.ds(ob, LANE)] = (
                acc + bi[pl.ds(ob, LANE)] + bj[pl.ds(ob, LANE)]
            )
            return 0

        lax.fori_loop(0, GROUPS, group, 0)

    pltpu.sync_copy(outv, out_hbm.at[pl.ds(base, BPW)])


def kernel(i, j, embedding, context_embedding, bias, context_bias):
    return _glove_sc(
        jnp.asarray(i, jnp.int32),
        jnp.asarray(j, jnp.int32),
        embedding,
        context_embedding,
        bias,
        context_bias,
    )


# speedup vs baseline: 1.4316x; 1.2032x over previous
import functools

import jax
import jax.numpy as jnp
from jax import lax
from jax.experimental import pallas as pl
from jax.experimental.pallas import tpu as pltpu
from jax.experimental.pallas import tpu_sc as plsc

VOCAB = 100000
DIM = 128
B = 16384
NC = 2
NS = 16
NW = NC * NS
BPW = B // NW
CHUNK = 128
NCHUNK = BPW // CHUNK
LANE = 16
GROUPS = CHUNK // LANE


def _mesh():
    return plsc.VectorSubcoreMesh(
        core_axis_name="c", subcore_axis_name="s", num_cores=NC, num_subcores=NS
    )


@functools.partial(
    pl.kernel,
    out_type=jax.ShapeDtypeStruct((B,), jnp.float32),
    mesh=_mesh(),
    scratch_types=[
        pltpu.VMEM((BPW,), jnp.int32),
        pltpu.VMEM((BPW,), jnp.int32),
        pltpu.VMEM((BPW,), jnp.float32),
        pltpu.VMEM((BPW,), jnp.float32),
        pltpu.VMEM((CHUNK, DIM), jnp.float32),
        pltpu.VMEM((CHUNK, DIM), jnp.float32),
        pltpu.VMEM((CHUNK, DIM), jnp.float32),
        pltpu.VMEM((CHUNK, DIM), jnp.float32),
        pltpu.VMEM((LANE * LANE,), jnp.float32),
        pltpu.VMEM((BPW,), jnp.float32),
        pltpu.SemaphoreType.DMA,
        pltpu.SemaphoreType.DMA,
        pltpu.SemaphoreType.DMA,
        pltpu.SemaphoreType.DMA,
    ],
)
def _glove_sc(i_hbm, j_hbm, emb_hbm, ctx_hbm, bias_hbm, cbias_hbm, out_hbm,
              idx_i, idx_j, bi, bj, wi0, wj0, wi1, wj1, srow, outv,
              sem_w0, sem_w1, sem_bi, sem_bj):
    wid = lax.axis_index("s") * NC + lax.axis_index("c")
    base = wid * BPW

    pltpu.sync_copy(i_hbm.at[pl.ds(base, BPW)], idx_i)
    pltpu.sync_copy(j_hbm.at[pl.ds(base, BPW)], idx_j)

    wbufs = [(wi0, wj0), (wi1, wj1)]
    wsems = [sem_w0, sem_w1]

    def fire_chunk(c):
        wi_b, wj_b = wbufs[c % 2]
        sem = wsems[c % 2]
        off = c * CHUNK
        ci = pltpu.async_copy(emb_hbm.at[idx_i.at[pl.ds(off, CHUNK)]], wi_b, sem)
        cj = pltpu.async_copy(ctx_hbm.at[idx_j.at[pl.ds(off, CHUNK)]], wj_b, sem)
        return ci, cj

    w_copies = [fire_chunk(0)]
    b_copies = []
    for c in range(NCHUNK):
        off = c * CHUNK
        cb_i = pltpu.async_copy(
            bias_hbm.at[idx_i.at[pl.ds(off, CHUNK)]], bi.at[pl.ds(off, CHUNK)],
            sem_bi)
        cb_j = pltpu.async_copy(
            cbias_hbm.at[idx_j.at[pl.ds(off, CHUNK)]], bj.at[pl.ds(off, CHUNK)],
            sem_bj)
        b_copies.append((cb_i, cb_j))

    lane = lax.iota(jnp.int32, LANE)
    perms = [lane ^ k for k in (8, 4, 2, 1)]
    diag = lane * (LANE + 1)

    for c in range(NCHUNK):
        wi_b, wj_b = wbufs[c % 2]
        ci, cj = w_copies[c]
        ci.wait()
        cj.wait()
        if c + 1 < NCHUNK:
            w_copies.append(fire_chunk(c + 1))
        cb_i, cb_j = b_copies[c]
        cb_i.wait()
        cb_j.wait()
        off = c * CHUNK

        def group(g, _, wi_b=wi_b, wj_b=wj_b, off=off):
            rbase = g * LANE
            for p in range(LANE):
                r = rbase + p
                prods = [
                    wi_b[r, pl.ds(k * LANE, LANE)] * wj_b[r, pl.ds(k * LANE, LANE)]
                    for k in range(DIM // LANE)
                ]
                while len(prods) > 1:
                    prods = [
                        prods[m] + prods[m + 1] for m in range(0, len(prods), 2)
                    ]
                s = prods[0]
                for perm in perms:
                    s = s + s.at[perm].get(mode="promise_in_bounds")
                srow[pl.ds(p * LANE, LANE)] = s
            res = jnp.zeros((LANE,), jnp.float32)
            for p in range(LANE):
                res = jnp.where(lane == p, srow[pl.ds(p * LANE, LANE)], res)
            ob = off + rbase
            outv[pl.ds(ob, LANE)] = (
                res + bi[pl.ds(ob, LANE)] + bj[pl.ds(ob, LANE)]
            )
            return 0

        lax.fori_loop(0, GROUPS, group, 0)

    pltpu.sync_copy(outv, out_hbm.at[pl.ds(base, BPW)])


def kernel(i, j, embedding, context_embedding, bias, context_bias):
    return _glove_sc(
        jnp.asarray(i, jnp.int32),
        jnp.asarray(j, jnp.int32),
        embedding,
        context_embedding,
        bias,
        context_bias,
    )


# speedup vs baseline: 1.8311x; 1.2791x over previous
import functools

import jax
import jax.numpy as jnp
from jax import lax
from jax.experimental import pallas as pl
from jax.experimental.pallas import tpu as pltpu
from jax.experimental.pallas import tpu_sc as plsc

VOCAB = 100000
DIM = 128
B = 16384
NC = 2
NS = 16
NW = NC * NS
BPW = B // NW
CHUNK = 128
NCHUNK = BPW // CHUNK
LANE = 16
GROUPS = CHUNK // LANE


def _mesh():
    return plsc.VectorSubcoreMesh(
        core_axis_name="c", subcore_axis_name="s", num_cores=NC, num_subcores=NS
    )


@functools.partial(
    pl.kernel,
    out_type=jax.ShapeDtypeStruct((B,), jnp.float32),
    mesh=_mesh(),
    scratch_types=[
        pltpu.VMEM((BPW,), jnp.int32),
        pltpu.VMEM((BPW,), jnp.int32),
        pltpu.VMEM((BPW,), jnp.float32),
        pltpu.VMEM((BPW,), jnp.float32),
        pltpu.VMEM((CHUNK, DIM), jnp.float32),
        pltpu.VMEM((CHUNK, DIM), jnp.float32),
        pltpu.VMEM((CHUNK, DIM), jnp.float32),
        pltpu.VMEM((CHUNK, DIM), jnp.float32),
        pltpu.VMEM((BPW,), jnp.float32),
        pltpu.SemaphoreType.DMA,
        pltpu.SemaphoreType.DMA,
        pltpu.SemaphoreType.DMA,
        pltpu.SemaphoreType.DMA,
    ],
)
def _glove_sc(i_hbm, j_hbm, emb_hbm, ctx_hbm, bias_hbm, cbias_hbm, out_hbm,
              idx_i, idx_j, bi, bj, wi0, wj0, wi1, wj1, outv,
              sem_w0, sem_w1, sem_bi, sem_bj):
    wid = lax.axis_index("s") * NC + lax.axis_index("c")
    base = wid * BPW

    pltpu.sync_copy(i_hbm.at[pl.ds(base, BPW)], idx_i)
    pltpu.sync_copy(j_hbm.at[pl.ds(base, BPW)], idx_j)

    wbufs = [(wi0, wj0), (wi1, wj1)]
    wsems = [sem_w0, sem_w1]

    def fire_chunk(c):
        wi_b, wj_b = wbufs[c % 2]
        sem = wsems[c % 2]
        off = c * CHUNK
        ci = pltpu.async_copy(emb_hbm.at[idx_i.at[pl.ds(off, CHUNK)]], wi_b, sem)
        cj = pltpu.async_copy(ctx_hbm.at[idx_j.at[pl.ds(off, CHUNK)]], wj_b, sem)
        return ci, cj

    w_copies = [fire_chunk(0)]
    b_copies = []
    for c in range(NCHUNK):
        off = c * CHUNK
        cb_i = pltpu.async_copy(
            bias_hbm.at[idx_i.at[pl.ds(off, CHUNK)]], bi.at[pl.ds(off, CHUNK)],
            sem_bi)
        cb_j = pltpu.async_copy(
            cbias_hbm.at[idx_j.at[pl.ds(off, CHUNK)]], bj.at[pl.ds(off, CHUNK)],
            sem_bj)
        b_copies.append((cb_i, cb_j))

    lane = lax.iota(jnp.int32, LANE)
    perm = {k: lane ^ k for k in (8, 4, 2, 1)}
    mask = {k: (lane & k) == 0 for k in (8, 4, 2, 1)}
    bitrev = [0, 8, 4, 12, 2, 10, 6, 14, 1, 9, 5, 13, 3, 11, 7, 15]

    def shuffle(v, k):
        return v.at[perm[k]].get(mode="promise_in_bounds")

    def merge(a, b, k):
        x = jnp.where(mask[k], a, shuffle(b, k))
        y = jnp.where(mask[k], shuffle(a, k), b)
        return x + y

    for c in range(NCHUNK):
        wi_b, wj_b = wbufs[c % 2]
        ci, cj = w_copies[c]
        ci.wait()
        cj.wait()
        if c + 1 < NCHUNK:
            w_copies.append(fire_chunk(c + 1))
        cb_i, cb_j = b_copies[c]
        cb_i.wait()
        cb_j.wait()
        off = c * CHUNK

        def group(g, _, wi_b=wi_b, wj_b=wj_b, off=off):
            rbase = g * LANE

            def kblock(k, accs, wi_b=wi_b, wj_b=wj_b, rbase=rbase):
                col = k * LANE
                return tuple(
                    accs[p]
                    + wi_b[rbase + p, pl.ds(col, LANE)]
                    * wj_b[rbase + p, pl.ds(col, LANE)]
                    for p in range(LANE)
                )

            zero = jnp.zeros((LANE,), jnp.float32)
            accs = lax.fori_loop(0, DIM // LANE, kblock, (zero,) * LANE)
            stack = []
            for p in bitrev:
                node = (16, accs[p])
                while stack and stack[-1][0] == node[0]:
                    lvl, other = stack.pop()
                    node = (lvl // 2, merge(other, node[1], lvl // 2))
                stack.append(node)
            res = stack[0][1]
            ob = off + rbase
            outv[pl.ds(ob, LANE)] = (
                res + bi[pl.ds(ob, LANE)] + bj[pl.ds(ob, LANE)]
            )
            return 0

        lax.fori_loop(0, GROUPS, group, 0)

    pltpu.sync_copy(outv, out_hbm.at[pl.ds(base, BPW)])


def kernel(i, j, embedding, context_embedding, bias, context_bias):
    return _glove_sc(
        jnp.asarray(i, jnp.int32),
        jnp.asarray(j, jnp.int32),
        embedding,
        context_embedding,
        bias,
        context_bias,
    )
